# Initial kernel scaffold; baseline (speedup 1.0000x reference)
import jax
import jax.numpy as jnp
from jax.experimental import pallas as pl


def kernel(superimposed_atom_repr, edge_index, r_labels, p_labels, r_floats, p_floats, emb0, emb1, emb2, W_rbf, b_rbf, W1, b1, W2, b2):
    raise NotImplementedError("write your pallas kernel here")


# speedup vs baseline: 5.5695x; 5.5695x over previous
import functools

import jax
import jax.numpy as jnp
from jax import lax
from jax.experimental import pallas as pl
from jax.experimental.pallas import tpu as pltpu
from jax.experimental.pallas import tpu_sc as plsc

N_NODES = 10000
N_EDGES = 320000
D = 128
NUM_RBF = 16
RBF_GAMMA = 10.0

NW = 32
NSEC = 5
NCHUNK = 25
CHUNK = 80
ACC_ROWS = 10240
ROWS_PER_TILE = ACC_ROWS // 16


def _bond_body(labs_r, labs_p, x_r, x_p, w128, bias, out_ref):
    eb = out_ref.shape[0]
    c = lax.broadcasted_iota(jnp.int32, (eb, D), 1)
    oh = jnp.zeros((eb, D), jnp.float32)
    for f in range(3):
        oh = oh + (labs_r[:, f][:, None] + (8 * f) == c).astype(jnp.float32)
        oh = oh + (labs_p[:, f][:, None] + (40 + 8 * f) == c).astype(jnp.float32)
    cf = c.astype(jnp.float32)
    xr = x_r[:, 0][:, None]
    xp = x_p[:, 0][:, None]
    d_r = xr - (cf - 24.0) / (NUM_RBF - 1.0)
    d_p = xp - (cf - 64.0) / (NUM_RBF - 1.0)
    v_r = jnp.exp(-RBF_GAMMA * d_r * d_r)
    v_p = jnp.exp(-RBF_GAMMA * d_p * d_p)
    m_r = jnp.logical_and(c >= 24, c < 24 + NUM_RBF)
    m_p = jnp.logical_and(c >= 64, c < 64 + NUM_RBF)
    feat = oh + jnp.where(m_r, v_r, 0.0) + jnp.where(m_p, v_p, 0.0)
    out_ref[...] = (
        jnp.dot(feat, w128[...], preferred_element_type=jnp.float32) + bias[...]
    )


def _bond_embed_all(r_labels, p_labels, r_floats, p_floats, w128, bias):
    EB = 2000
    grid = N_EDGES // EB

    def body(lr, lp, xr, xp, w, b, o):
        _bond_body(lr[...], lp[...], xr[...], xp[...], w, b, o)

    return pl.pallas_call(
        body,
        grid=(grid,),
        in_specs=[
            pl.BlockSpec((EB, 3), lambda i: (i, 0)),
            pl.BlockSpec((EB, 3), lambda i: (i, 0)),
            pl.BlockSpec((EB, 1), lambda i: (i, 0)),
            pl.BlockSpec((EB, 1), lambda i: (i, 0)),
            pl.BlockSpec((D, D), lambda i: (0, 0)),
            pl.BlockSpec((1, D), lambda i: (0, 0)),
        ],
        out_specs=pl.BlockSpec((EB, D), lambda i: (i, 0)),
        out_shape=jax.ShapeDtypeStruct((N_EDGES, D), jnp.float32),
    )(r_labels, p_labels, r_floats, p_floats, w128, bias)


def _sc_scatter(node_repr, src3, dst3, bond4):
    mesh = plsc.VectorSubcoreMesh(core_axis_name="c", subcore_axis_name="s")

    @functools.partial(
        pl.kernel,
        mesh=mesh,
        out_type=jax.ShapeDtypeStruct((2, ACC_ROWS, D), jnp.float32),
        scratch_types=[
            pltpu.VMEM((NCHUNK, CHUNK), jnp.int32),
            pltpu.VMEM((NCHUNK, CHUNK), jnp.int32),
            pltpu.VMEM((CHUNK, D), jnp.float32),
            pltpu.VMEM((CHUNK, D), jnp.float32),
            pltpu.VMEM_SHARED((ACC_ROWS, D), jnp.float32),
            pltpu.SemaphoreType.DMA,
        ],
    )
    def k(a_hbm, src_hbm, dst_hbm, bond_hbm, out_hbm,
          src_v, dst_v, rows_v, bond_v, acc, sem):
        cid = lax.axis_index("c")
        sid = lax.axis_index("s")
        wid = sid * 2 + cid

        def zloop(i, _):
            rows_v[i // 8, pl.ds((i % 8) * 16, 16)] = jnp.zeros((16,), jnp.float32)
            return _
        lax.fori_loop(0, CHUNK * (D // 16), zloop, None)
        for t in range(ROWS_PER_TILE // CHUNK):
            pltpu.sync_copy(
                rows_v, acc.at[pl.ds(sid * ROWS_PER_TILE + t * CHUNK, CHUNK)]
            )
        plsc.subcore_barrier()

        for s in range(NSEC):
            pltpu.sync_copy(src_hbm.at[wid, s], src_v)
            pltpu.sync_copy(dst_hbm.at[wid, s], dst_v)

            def chunk(j, _):
                pltpu.async_copy(a_hbm.at[src_v.at[j]], rows_v, sem).wait()
                pltpu.sync_copy(bond_hbm.at[wid, s, j], bond_v)
                pltpu.sync_copy(rows_v, acc.at[dst_v.at[j]], add=True)
                pltpu.sync_copy(bond_v, acc.at[dst_v.at[j]], add=True)
                return _
            lax.fori_loop(0, NCHUNK, chunk, None)

        plsc.subcore_barrier()
        pltpu.sync_copy(
            acc.at[pl.ds(sid * ROWS_PER_TILE, ROWS_PER_TILE)],
            out_hbm.at[cid, pl.ds(sid * ROWS_PER_TILE, ROWS_PER_TILE)],
        )

    return k(node_repr, src3, dst3, bond4)


def _mlp(parts, W1, b1, W2, b2):
    NB = 2048
    NROWS = parts.shape[1]

    def body(p_ref, w1, b1r, w2, b2r, o_ref):
        agg = p_ref[0] + p_ref[1]
        h = jnp.maximum(
            jnp.dot(agg, w1[...], preferred_element_type=jnp.float32) + b1r[...], 0.0
        )
        o_ref[...] = jnp.maximum(
            jnp.dot(h, w2[...], preferred_element_type=jnp.float32) + b2r[...], 0.0
        )

    return pl.pallas_call(
        body,
        grid=(NROWS // NB,),
        in_specs=[
            pl.BlockSpec((2, NB, D), lambda i: (0, i, 0)),
            pl.BlockSpec((D, 2 * D), lambda i: (0, 0)),
            pl.BlockSpec((1, 2 * D), lambda i: (0, 0)),
            pl.BlockSpec((2 * D, D), lambda i: (0, 0)),
            pl.BlockSpec((1, D), lambda i: (0, 0)),
        ],
        out_specs=pl.BlockSpec((NB, D), lambda i: (i, 0)),
        out_shape=jax.ShapeDtypeStruct((NROWS, D), jnp.float32),
    )(parts, W1, b1, W2, b2)


def kernel(superimposed_atom_repr, edge_index, r_labels, p_labels, r_floats,
           p_floats, emb0, emb1, emb2, W_rbf, b_rbf, W1, b1, W2, b2):
    Wc = jnp.concatenate([emb0, emb1, emb2], axis=0)
    top = jnp.concatenate(
        [jnp.concatenate([Wc, -Wc], 1), jnp.concatenate([W_rbf, -W_rbf], 1)], 0
    )
    bot = jnp.concatenate(
        [jnp.concatenate([jnp.zeros_like(Wc), Wc], 1),
         jnp.concatenate([jnp.zeros_like(W_rbf), W_rbf], 1)], 0
    )
    w128 = jnp.concatenate([top, bot, jnp.zeros((48, D), jnp.float32)], 0)
    bias = jnp.concatenate([b_rbf, jnp.zeros((64,), jnp.float32)]).reshape(1, D)

    bond = _bond_embed_all(r_labels, p_labels, r_floats, p_floats, w128, bias)

    src3 = edge_index[0].reshape(NW, NSEC, NCHUNK, CHUNK)
    dst3 = edge_index[1].reshape(NW, NSEC, NCHUNK, CHUNK)
    bond4 = bond.reshape(NW, NSEC, NCHUNK, CHUNK, D)

    parts = _sc_scatter(superimposed_atom_repr, src3, dst3, bond4)

    out = _mlp(parts, W1, b1.reshape(1, 2 * D), W2, b2.reshape(1, D))
    return out[:N_NODES]


# speedup vs baseline: 10.2429x; 1.8391x over previous
import functools

import jax
import jax.numpy as jnp
from jax import lax
from jax.experimental import pallas as pl
from jax.experimental.pallas import tpu as pltpu
from jax.experimental.pallas import tpu_sc as plsc

N_NODES = 10000
N_EDGES = 320000
D = 128
NUM_RBF = 16
RBF_GAMMA = 10.0

NW = 32
NSEC = 5
NCHUNK = 25
CHUNK = 80
ACC_ROWS = 10240
ROWS_PER_TILE = ACC_ROWS // 16


def _rbf_all(r_floats, p_floats):
    def body(xr_ref, xp_ref, o_ref):
        xr = xr_ref[0, 0]
        xp = xp_ref[0, 0]
        c = lax.broadcasted_iota(jnp.int32, (NCHUNK, CHUNK, 2 * NUM_RBF), 2)
        center = (c % NUM_RBF).astype(jnp.float32) / (NUM_RBF - 1.0)
        x = jnp.where(c < NUM_RBF, xr[:, :, None], xp[:, :, None])
        d = x - center
        o_ref[0, 0] = jnp.exp(-RBF_GAMMA * d * d)

    return pl.pallas_call(
        body,
        grid=(NW, NSEC),
        in_specs=[
            pl.BlockSpec((1, 1, NCHUNK, CHUNK), lambda i, j: (i, j, 0, 0)),
            pl.BlockSpec((1, 1, NCHUNK, CHUNK), lambda i, j: (i, j, 0, 0)),
        ],
        out_specs=pl.BlockSpec(
            (1, 1, NCHUNK, CHUNK, 2 * NUM_RBF), lambda i, j: (i, j, 0, 0, 0)
        ),
        out_shape=jax.ShapeDtypeStruct(
            (NW, NSEC, NCHUNK, CHUNK, 2 * NUM_RBF), jnp.float32
        ),
    )(r_floats, p_floats)


def _sc_gather_scatter(node_repr, src3, dst3):
    mesh = plsc.VectorSubcoreMesh(core_axis_name="c", subcore_axis_name="s")

    @functools.partial(
        pl.kernel,
        mesh=mesh,
        out_type=jax.ShapeDtypeStruct((2, ACC_ROWS, D), jnp.float32),
        scratch_types=[
            pltpu.VMEM((NCHUNK, CHUNK), jnp.int32),
            pltpu.VMEM((NCHUNK, CHUNK), jnp.int32),
            pltpu.VMEM((CHUNK, D), jnp.float32),
            pltpu.VMEM((CHUNK, D), jnp.float32),
            pltpu.VMEM_SHARED((ACC_ROWS, D), jnp.float32),
            pltpu.SemaphoreType.DMA,
            pltpu.SemaphoreType.DMA,
        ],
    )
    def k(a_hbm, src_hbm, dst_hbm, out_hbm,
          src_v, dst_v, rows0, rows1, acc, sem0, sem1):
        cid = lax.axis_index("c")
        sid = lax.axis_index("s")
        wid = sid * 2 + cid

        def zloop(i, _):
            rows0[i // 8, pl.ds((i % 8) * 16, 16)] = jnp.zeros((16,), jnp.float32)
            return _
        lax.fori_loop(0, CHUNK * (D // 16), zloop, None)
        for t in range(ROWS_PER_TILE // CHUNK):
            pltpu.sync_copy(
                rows0, acc.at[pl.ds(sid * ROWS_PER_TILE + t * CHUNK, CHUNK)]
            )
        plsc.subcore_barrier()

        bufs = (rows0, rows1)
        sems = (sem0, sem1)
        for s in range(NSEC):
            pltpu.sync_copy(src_hbm.at[wid, s], src_v)
            pltpu.sync_copy(dst_hbm.at[wid, s], dst_v)
            d0 = pltpu.async_copy(a_hbm.at[src_v.at[0]], rows0, sem0)

            def chunk(j, _):
                @pl.when(j + 1 < NCHUNK)
                def _issue():
                    for b in range(2):
                        @pl.when(lax.rem(j + 1, 2) == b)
                        def _():
                            pltpu.async_copy(
                                a_hbm.at[src_v.at[j + 1]], bufs[b], sems[b]
                            )
                for b in range(2):
                    @pl.when(lax.rem(j, 2) == b)
                    def _():
                        pltpu.make_async_copy(
                            a_hbm.at[src_v.at[j]], bufs[b], sems[b]
                        ).wait()
                        pltpu.sync_copy(bufs[b], acc.at[dst_v.at[j]], add=True)
                return _
            lax.fori_loop(0, NCHUNK, chunk, None)

        plsc.subcore_barrier()
        pltpu.sync_copy(
            acc.at[pl.ds(sid * ROWS_PER_TILE, ROWS_PER_TILE)],
            out_hbm.at[cid, pl.ds(sid * ROWS_PER_TILE, ROWS_PER_TILE)],
        )

    return k(node_repr, src3, dst3)


def _sc_aux(labsT4, rbf5, dst3):
    mesh = plsc.VectorSubcoreMesh(core_axis_name="c", subcore_axis_name="s")

    @functools.partial(
        pl.kernel,
        mesh=mesh,
        out_type=jax.ShapeDtypeStruct((2, ACC_ROWS, D), jnp.float32),
        scratch_types=[
            pltpu.VMEM((NCHUNK, CHUNK), jnp.int32),
            pltpu.VMEM((CHUNK, 16), jnp.int32),
            pltpu.VMEM((CHUNK, 2 * NUM_RBF), jnp.float32),
            pltpu.VMEM((CHUNK, D), jnp.float32),
            pltpu.VMEM_SHARED((ACC_ROWS, D), jnp.float32),
        ],
    )
    def k(labs_hbm, rbf_hbm, dst_hbm, out_hbm,
          dst_v, labs_v, rbf_v, aux, acc):
        cid = lax.axis_index("c")
        sid = lax.axis_index("s")
        wid = sid * 2 + cid

        def zloop(i, _):
            aux[i // 8, pl.ds((i % 8) * 16, 16)] = jnp.zeros((16,), jnp.float32)
            return _
        lax.fori_loop(0, CHUNK * (D // 16), zloop, None)
        for t in range(ROWS_PER_TILE // CHUNK):
            pltpu.sync_copy(
                aux, acc.at[pl.ds(sid * ROWS_PER_TILE + t * CHUNK, CHUNK)]
            )
        plsc.subcore_barrier()

        for s in range(NSEC):
            pltpu.sync_copy(dst_hbm.at[wid, s], dst_v)

            def chunk(j, _):
                pltpu.sync_copy(labs_hbm.at[wid, s, j], labs_v)
                pltpu.sync_copy(rbf_hbm.at[wid, s, j], rbf_v)

                def rloop(r, _2):
                    io = lax.iota(jnp.int32, 16)
                    lv = labs_v[r]
                    one = jnp.ones((16,), jnp.float32)
                    zero = jnp.zeros((16,), jnp.float32)
                    aux[r, pl.ds(0, 16)] = jnp.where(
                        io == lv[0], one, zero
                    ) + jnp.where(io == lv[1] + 8, one, zero)
                    aux[r, pl.ds(16, 16)] = jnp.where(
                        io == lv[2], one, zero
                    ) + jnp.where(io == lv[3] + 8, one, zero)
                    aux[r, pl.ds(32, 16)] = jnp.where(
                        io == lv[4], one, zero
                    ) + jnp.where(io == lv[5] + 8, one, zero)
                    aux[r, pl.ds(48, 16)] = rbf_v[r, pl.ds(0, 16)]
                    aux[r, pl.ds(64, 16)] = rbf_v[r, pl.ds(16, 16)]
                    return _2
                lax.fori_loop(0, CHUNK, rloop, None)

                pltpu.sync_copy(aux, acc.at[dst_v.at[j]], add=True)
                return _
            lax.fori_loop(0, NCHUNK, chunk, None)

        plsc.subcore_barrier()
        pltpu.sync_copy(
            acc.at[pl.ds(sid * ROWS_PER_TILE, ROWS_PER_TILE)],
            out_hbm.at[cid, pl.ds(sid * ROWS_PER_TILE, ROWS_PER_TILE)],
        )

    return k(labsT4, rbf5, dst3)


def _mlp(pA, pX, Waux, W1, b1, W2, b2):
    NB = 2048
    NROWS = pA.shape[1]

    def body(pa_ref, px_ref, wa, w1, b1r, w2, b2r, o_ref):
        agg = pa_ref[0] + pa_ref[1]
        aux = px_ref[0] + px_ref[1]
        agg = agg + jnp.dot(aux, wa[...], preferred_element_type=jnp.float32)
        h = jnp.maximum(
            jnp.dot(agg, w1[...], preferred_element_type=jnp.float32) + b1r[...], 0.0
        )
        o_ref[...] = jnp.maximum(
            jnp.dot(h, w2[...], preferred_element_type=jnp.float32) + b2r[...], 0.0
        )

    return pl.pallas_call(
        body,
        grid=(NROWS // NB,),
        in_specs=[
            pl.BlockSpec((2, NB, D), lambda i: (0, i, 0)),
            pl.BlockSpec((2, NB, D), lambda i: (0, i, 0)),
            pl.BlockSpec((D, D), lambda i: (0, 0)),
            pl.BlockSpec((D, 2 * D), lambda i: (0, 0)),
            pl.BlockSpec((1, 2 * D), lambda i: (0, 0)),
            pl.BlockSpec((2 * D, D), lambda i: (0, 0)),
            pl.BlockSpec((1, D), lambda i: (0, 0)),
        ],
        out_specs=pl.BlockSpec((NB, D), lambda i: (i, 0)),
        out_shape=jax.ShapeDtypeStruct((NROWS, D), jnp.float32),
    )(pA, pX, Waux, W1, b1, W2, b2)


def kernel(superimposed_atom_repr, edge_index, r_labels, p_labels, r_floats,
           p_floats, emb0, emb1, emb2, W_rbf, b_rbf, W1, b1, W2, b2):
    Wc = jnp.concatenate([emb0, emb1, emb2], axis=0)
    z64 = jnp.zeros((64,), jnp.float32)
    bias_row = jnp.concatenate([b_rbf, z64]).reshape(1, D)
    r_rows = jnp.concatenate([Wc, -Wc], 1)
    r_rows = r_rows.at[0:8].add(bias_row)
    p_rows = jnp.concatenate([jnp.zeros_like(Wc), Wc], 1)
    rbf_r_rows = jnp.concatenate([W_rbf, -W_rbf], 1)
    rbf_p_rows = jnp.concatenate([jnp.zeros_like(W_rbf), W_rbf], 1)
    Waux = jnp.concatenate(
        [r_rows, p_rows, rbf_r_rows, rbf_p_rows,
         jnp.zeros((48, D), jnp.float32)], 0
    )

    src3 = edge_index[0].reshape(NW, NSEC, NCHUNK, CHUNK)
    dst3 = edge_index[1].reshape(NW, NSEC, NCHUNK, CHUNK)
    labs16 = jnp.concatenate(
        [r_labels, p_labels, jnp.zeros((N_EDGES, 10), jnp.int32)], axis=1
    ).reshape(NW, NSEC, NCHUNK, CHUNK, 16)

    xr4 = r_floats.reshape(NW, NSEC, NCHUNK, CHUNK)
    xp4 = p_floats.reshape(NW, NSEC, NCHUNK, CHUNK)
    rbf5 = _rbf_all(xr4, xp4)

    pA = _sc_gather_scatter(superimposed_atom_repr, src3, dst3)
    pX = _sc_aux(labs16, rbf5, dst3)

    out = _mlp(pA, pX, Waux, W1, b1.reshape(1, 2 * D), W2, b2.reshape(1, D))
    return out[:N_NODES]
